# Initial kernel scaffold; baseline (speedup 1.0000x reference)
import jax
import jax.numpy as jnp
from jax.experimental import pallas as pl


def kernel(nodes, edge_attr, senders, receivers, train, params):
    raise NotImplementedError("write your pallas kernel here")


# speedup vs baseline: 2.9005x; 2.9005x over previous
import functools

import jax
import jax.numpy as jnp
from jax import lax
from jax.experimental import pallas as pl
from jax.experimental.pallas import tpu as pltpu
from jax.experimental.pallas import tpu_sc as plsc

N = 10000
E = 320000
D = 128

NC = 2
NS = 16
NW = NC * NS

NPAD = 10240
STRIPE = NPAD // NS

CHUNK = 80
EPT = E // NW
EPS = E // NS

_MESH = plsc.VectorSubcoreMesh(
    core_axis_name="c", subcore_axis_name="s", num_cores=NC, num_subcores=NS)


def _mm(x, w, c, br=2000):
    r, k = x.shape
    dout = w.shape[1]

    def body(x_ref, w_ref, c_ref, o_ref):
        o_ref[...] = (
            jnp.dot(x_ref[...], w_ref[...], preferred_element_type=jnp.float32)
            + c_ref[...])

    return pl.pallas_call(
        body,
        grid=(r // br,),
        in_specs=[
            pl.BlockSpec((br, k), lambda i: (i, 0)),
            pl.BlockSpec((k, dout), lambda i: (0, 0)),
            pl.BlockSpec((1, dout), lambda i: (0, 0)),
        ],
        out_specs=pl.BlockSpec((br, dout), lambda i: (i, 0)),
        out_shape=jax.ShapeDtypeStruct((r, dout), jnp.float32),
    )(x, w, c)


def _ln_relu(x, s, b):
    m = jnp.mean(x, axis=-1, keepdims=True)
    xc = x - m
    v = jnp.mean(xc * xc, axis=-1, keepdims=True)
    return jax.nn.relu(xc * lax.rsqrt(v + 1e-6) * s + b)


def _edge_finish(m_arr, ga, gb, s, b, br=2000):
    def body(m_ref, a_ref, b_ref, s_ref, bb_ref, o_ref):
        x = m_ref[...] + a_ref[...] + b_ref[...]
        o_ref[...] = _ln_relu(x, s_ref[...], bb_ref[...])

    return pl.pallas_call(
        body,
        grid=(E // br,),
        in_specs=[
            pl.BlockSpec((br, D), lambda i: (i, 0)),
            pl.BlockSpec((br, D), lambda i: (i, 0)),
            pl.BlockSpec((br, D), lambda i: (i, 0)),
            pl.BlockSpec((1, D), lambda i: (0, 0)),
            pl.BlockSpec((1, D), lambda i: (0, 0)),
        ],
        out_specs=pl.BlockSpec((br, D), lambda i: (i, 0)),
        out_shape=jax.ShapeDtypeStruct((E, D), jnp.float32),
    )(m_arr, ga, gb, s, b)


def _node_pre(h_n, w_a, w_b, w_t):
    def body(x_ref, wa_ref, wb_ref, wt_ref, a_ref, b_ref, t_ref):
        x = x_ref[...]
        a_ref[...] = jnp.dot(x, wa_ref[...], preferred_element_type=jnp.float32)
        b_ref[...] = jnp.dot(x, wb_ref[...], preferred_element_type=jnp.float32)
        t_ref[...] = jnp.dot(x, wt_ref[...], preferred_element_type=jnp.float32)

    br = 2000
    sds = jax.ShapeDtypeStruct((N, D), jnp.float32)
    return pl.pallas_call(
        body,
        grid=(N // br,),
        in_specs=[
            pl.BlockSpec((br, D), lambda i: (i, 0)),
            pl.BlockSpec((D, D), lambda i: (0, 0)),
            pl.BlockSpec((D, D), lambda i: (0, 0)),
            pl.BlockSpec((D, D), lambda i: (0, 0)),
        ],
        out_specs=[
            pl.BlockSpec((br, D), lambda i: (i, 0)),
            pl.BlockSpec((br, D), lambda i: (i, 0)),
            pl.BlockSpec((br, D), lambda i: (i, 0)),
        ],
        out_shape=[sds, sds, sds],
    )(h_n, w_a, w_b, w_t)


def _node_update(t, sent, recv, w_s, w_r, c, s, b):
    br = 2000

    def body(t_ref, sp_ref, rp_ref, ws_ref, wr_ref, c_ref, s_ref, b_ref,
             o_ref, nsum_ref, esum_ref):
        i = pl.program_id(0)
        sent_blk = sp_ref[...]
        recv_blk = rp_ref[...]
        x = (t_ref[...]
             + jnp.dot(sent_blk, ws_ref[...], preferred_element_type=jnp.float32)
             + jnp.dot(recv_blk, wr_ref[...], preferred_element_type=jnp.float32)
             + c_ref[...])
        h = _ln_relu(x, s_ref[...], b_ref[...])
        o_ref[...] = h

        @pl.when(i == 0)
        def _():
            nsum_ref[...] = jnp.zeros_like(nsum_ref)
            esum_ref[...] = jnp.zeros_like(esum_ref)

        nsum_ref[...] += jnp.sum(h, axis=0, keepdims=True)
        esum_ref[...] += jnp.sum(sent_blk, axis=0, keepdims=True)

    one = jax.ShapeDtypeStruct((1, D), jnp.float32)
    return pl.pallas_call(
        body,
        grid=(N // br,),
        in_specs=[
            pl.BlockSpec((br, D), lambda i: (i, 0)),
            pl.BlockSpec((br, D), lambda i: (i, 0)),
            pl.BlockSpec((br, D), lambda i: (i, 0)),
            pl.BlockSpec((D, D), lambda i: (0, 0)),
            pl.BlockSpec((D, D), lambda i: (0, 0)),
            pl.BlockSpec((1, D), lambda i: (0, 0)),
            pl.BlockSpec((1, D), lambda i: (0, 0)),
            pl.BlockSpec((1, D), lambda i: (0, 0)),
        ],
        out_specs=[
            pl.BlockSpec((br, D), lambda i: (i, 0)),
            pl.BlockSpec((1, D), lambda i: (0, 0)),
            pl.BlockSpec((1, D), lambda i: (0, 0)),
        ],
        out_shape=[jax.ShapeDtypeStruct((N, D), jnp.float32), one, one],
    )(t, sent, recv, w_s, w_r, c, s, b)


def _global_update(nsum, esum, g, wg, bg, lns, lnb, w_e_g, be, w_n_g, bn):
    def body(ns_ref, es_ref, g_ref, wg_ref, bg_ref, s_ref, b_ref,
             weg_ref, be_ref, wng_ref, bn_ref, g_out, ce_out, cn_out):
        wg = wg_ref[...]
        x = (jnp.dot(ns_ref[...], wg[0:D, :], preferred_element_type=jnp.float32)
             + jnp.dot(es_ref[...], wg[D:2 * D, :], preferred_element_type=jnp.float32)
             + jnp.dot(g_ref[...], wg[2 * D:3 * D, :], preferred_element_type=jnp.float32)
             + bg_ref[...])
        gn = _ln_relu(x, s_ref[...], b_ref[...])
        g_out[...] = gn
        ce_out[...] = jnp.dot(gn, weg_ref[...], preferred_element_type=jnp.float32) + be_ref[...]
        cn_out[...] = jnp.dot(gn, wng_ref[...], preferred_element_type=jnp.float32) + bn_ref[...]

    one = jax.ShapeDtypeStruct((1, D), jnp.float32)
    return pl.pallas_call(
        body,
        out_shape=[one, one, one],
    )(nsum, esum, g, wg, bg, lns, lnb, w_e_g, be, w_n_g, bn)


def _global_final(nsum, esum, g, wg, bg, lns, lnb, wdec, bdec):
    def body(ns_ref, es_ref, g_ref, wg_ref, bg_ref, s_ref, b_ref,
             wd_ref, bd_ref, o_ref):
        wg = wg_ref[...]
        x = (jnp.dot(ns_ref[...], wg[0:D, :], preferred_element_type=jnp.float32)
             + jnp.dot(es_ref[...], wg[D:2 * D, :], preferred_element_type=jnp.float32)
             + jnp.dot(g_ref[...], wg[2 * D:3 * D, :], preferred_element_type=jnp.float32)
             + bg_ref[...])
        gn = _ln_relu(x, s_ref[...], b_ref[...])
        o_ref[...] = jnp.dot(gn, wd_ref[...], preferred_element_type=jnp.float32) + bd_ref[...]

    return pl.pallas_call(
        body,
        out_shape=jax.ShapeDtypeStruct((1, D), jnp.float32),
    )(nsum, esum, g, wg, bg, lns, lnb, wdec, bdec)


def _sc_gather2(table_a, table_b, senders, receivers):
    @functools.partial(
        pl.kernel,
        out_type=[jax.ShapeDtypeStruct((E, D), jnp.float32),
                  jax.ShapeDtypeStruct((E, D), jnp.float32)],
        mesh=_MESH,
        scratch_types=[
            pltpu.VMEM((CHUNK,), jnp.int32),
            pltpu.VMEM((CHUNK,), jnp.int32),
            pltpu.VMEM((CHUNK, D), jnp.float32),
            pltpu.VMEM((CHUNK, D), jnp.float32),
            pltpu.SemaphoreType.DMA,
            pltpu.SemaphoreType.DMA,
        ],
    )
    def k(ta_hbm, tb_hbm, s_hbm, r_hbm, oa_hbm, ob_hbm,
          ia_v, ib_v, ra_v, rb_v, sem_a, sem_b):
        wid = lax.axis_index("s") * NC + lax.axis_index("c")
        base = wid * EPT

        def body(j, carry):
            off = base + j * CHUNK
            pltpu.sync_copy(s_hbm.at[pl.ds(off, CHUNK)], ia_v)
            pltpu.sync_copy(r_hbm.at[pl.ds(off, CHUNK)], ib_v)
            cp_a = pltpu.async_copy(ta_hbm.at[ia_v], ra_v, sem_a)
            cp_b = pltpu.async_copy(tb_hbm.at[ib_v], rb_v, sem_b)
            cp_a.wait()
            cp_b.wait()
            pltpu.sync_copy(ra_v, oa_hbm.at[pl.ds(off, CHUNK)])
            pltpu.sync_copy(rb_v, ob_hbm.at[pl.ds(off, CHUNK)])
            return carry

        lax.fori_loop(0, EPT // CHUNK, body, 0)

    return k(table_a, table_b, senders, receivers)


def _sc_segsum2(data, senders, receivers):
    @functools.partial(
        pl.kernel,
        out_type=[jax.ShapeDtypeStruct((NPAD, D), jnp.float32),
                  jax.ShapeDtypeStruct((NPAD, D), jnp.float32)],
        mesh=_MESH,
        scratch_types=[
            pltpu.VMEM((CHUNK,), jnp.int32),
            pltpu.VMEM((CHUNK, D), jnp.float32),
            pltpu.VMEM((CHUNK, D), jnp.float32),
            pltpu.VMEM_SHARED((NPAD, D), jnp.float32),
        ],
    )
    def k(d_hbm, s_hbm, r_hbm, sent_hbm, recv_hbm, idx_v, rows_v, zbuf, acc):
        core = lax.axis_index("c")
        sid = lax.axis_index("s")

        def zbody(kk, carry):
            i = kk // 8
            j = (kk % 8) * 16
            zbuf[i, pl.ds(j, 16)] = jnp.zeros((16,), jnp.float32)
            return carry

        lax.fori_loop(0, CHUNK * 8, zbody, 0)
        for t in range(STRIPE // CHUNK):
            pltpu.sync_copy(zbuf, acc.at[pl.ds(sid * STRIPE + t * CHUNK, CHUNK)])
        plsc.subcore_barrier()

        def make_body(idx_hbm):
            def body(j, carry):
                off = sid * EPS + j * CHUNK
                pltpu.sync_copy(idx_hbm.at[pl.ds(off, CHUNK)], idx_v)
                pltpu.sync_copy(d_hbm.at[pl.ds(off, CHUNK)], rows_v)
                pltpu.sync_copy(rows_v, acc.at[idx_v], add=True)
                return carry
            return body

        @pl.when(core == 0)
        def _():
            lax.fori_loop(0, EPS // CHUNK, make_body(s_hbm), 0)

        @pl.when(core == 1)
        def _():
            lax.fori_loop(0, EPS // CHUNK, make_body(r_hbm), 0)

        plsc.subcore_barrier()

        @pl.when(core == 0)
        def _():
            pltpu.sync_copy(acc.at[pl.ds(sid * STRIPE, STRIPE)],
                            sent_hbm.at[pl.ds(sid * STRIPE, STRIPE)])

        @pl.when(core == 1)
        def _():
            pltpu.sync_copy(acc.at[pl.ds(sid * STRIPE, STRIPE)],
                            recv_hbm.at[pl.ds(sid * STRIPE, STRIPE)])

    return k(data, senders, receivers)


def kernel(nodes, edge_attr, senders, receivers, train, params):
    del train
    senders = senders.astype(jnp.int32)
    receivers = receivers.astype(jnp.int32)

    h_n = _mm(nodes, params['en']['W'], params['en']['b'][None])
    h_e = _mm(edge_attr, params['ee']['W'], params['ee']['b'][None])
    g = jnp.zeros((1, D), jnp.float32)

    c_e = params['steps'][0]['e']['b'][None]
    c_n = params['steps'][0]['n']['b'][None]

    out = None
    for i, sp in enumerate(params['steps']):
        we = sp['e']['W']
        wn = sp['n']['W']

        a_tab, b_tab, t_arr = _node_pre(h_n, we[D:2 * D], we[2 * D:3 * D],
                                        wn[0:D])
        m_arr = _mm(h_e, we[0:D], c_e)
        ga, gb = _sc_gather2(a_tab, b_tab, senders, receivers)
        h_e = _edge_finish(m_arr, ga, gb, sp['e']['ln_s'][None],
                           sp['e']['ln_b'][None])
        sent, recv = _sc_segsum2(h_e, senders, receivers)
        h_n, nsum, esum = _node_update(
            t_arr, sent[:N], recv[:N], wn[D:2 * D], wn[2 * D:3 * D],
            c_n, sp['n']['ln_s'][None], sp['n']['ln_b'][None])

        gp = sp['g']
        if i + 1 < len(params['steps']):
            nxt = params['steps'][i + 1]
            g, c_e, c_n = _global_update(
                nsum, esum, g, gp['W'], gp['b'][None],
                gp['ln_s'][None], gp['ln_b'][None],
                nxt['e']['W'][3 * D:], nxt['e']['b'][None],
                nxt['n']['W'][3 * D:], nxt['n']['b'][None])
        else:
            out = _global_final(
                nsum, esum, g, gp['W'], gp['b'][None],
                gp['ln_s'][None], gp['ln_b'][None],
                params['dec']['W'], params['dec']['b'][None])

    return out
